# Initial kernel scaffold; baseline (speedup 1.0000x reference)
import jax
import jax.numpy as jnp
from jax.experimental import pallas as pl


def kernel(logits):
    raise NotImplementedError("write your pallas kernel here")


# speedup vs baseline: 28.7728x; 28.7728x over previous
import functools

import jax
import jax.numpy as jnp
from jax.experimental import pallas as pl
from jax.experimental.pallas import tpu as pltpu

_TEMP = 0.8
_TOPP = 0.9
_NT = 64
_R = 4
_W0 = 20.0
_C = 32768


def _colmask(j, shape, V):
    lane = jax.lax.broadcasted_iota(jnp.int32, shape, 1)
    return (j * _C + lane) < V


def _stats_body(x_ref, m_out, z_out, V):
    j = pl.program_id(0)
    x = jnp.where(_colmask(j, x_ref.shape, V), x_ref[...] / _TEMP, -jnp.inf)
    cm = jnp.max(x, axis=-1, keepdims=True)
    ce = jnp.sum(jnp.exp(x - cm), axis=-1, keepdims=True)

    @pl.when(j == 0)
    def _():
        m_out[...] = jnp.full_like(m_out, -jnp.inf)
        z_out[...] = jnp.zeros_like(z_out)

    m_old = m_out[...]
    m_new = jnp.maximum(m_old, cm)
    z_out[...] = z_out[...] * jnp.exp(m_old - m_new) + ce * jnp.exp(cm - m_new)
    m_out[...] = m_new


def _refine_body(x_ref, m_ref, z_ref, th_out, se_out, hi_s, w_s, acc_s, V):
    r = pl.program_id(0)
    j = pl.program_id(1)
    nj = pl.num_programs(1)

    @pl.when((r == 0) & (j == 0))
    def _():
        hi_s[...] = m_ref[...]
        w_s[...] = jnp.full_like(w_s, _W0 / _NT)

    @pl.when(j == 0)
    def _():
        acc_s[...] = jnp.zeros_like(acc_s)

    x = jnp.where(_colmask(j, x_ref.shape, V), x_ref[...] / _TEMP, -jnp.inf)
    e = jnp.exp(x - m_ref[...])
    hi = hi_s[...]
    w = w_s[...]
    lane = jax.lax.broadcasted_iota(jnp.int32, acc_s.shape, 1)
    inc = jnp.zeros(acc_s.shape, jnp.float32)
    for t in range(_NT):
        s_t = jnp.sum(jnp.where(x > (hi - (t + 1.0) * w), e, 0.0), axis=-1,
                      keepdims=True)
        inc = jnp.where(lane == t, s_t, inc)
    acc_s[...] += inc

    @pl.when(j == nj - 1)
    def _():
        acc = acc_s[...]
        target = _TOPP * z_ref[...]
        notc = jnp.where(acc > target, 0.0, 1.0)
        tstar_i = jnp.sum(notc, axis=-1, keepdims=True).astype(jnp.int32)
        tstar = tstar_i.astype(jnp.float32)
        th_out[...] = hi - (tstar + 1.0) * w
        lane_t = jax.lax.broadcasted_iota(jnp.int32, acc.shape, 1)
        se_out[...] = jnp.sum(
            jnp.where(lane_t == tstar_i, acc, 0.0), axis=-1, keepdims=True)
        hi_s[...] = hi - tstar * w
        w_s[...] = 2.0 * w / _NT


def _final_body(x_ref, g_ref, m_ref, z_ref, th_ref, se_ref,
                f_out, id_out, best_s, bidx_s, V):
    j = pl.program_id(0)

    @pl.when(j == 0)
    def _():
        best_s[...] = jnp.full_like(best_s, -jnp.inf)
        bidx_s[...] = jnp.zeros_like(bidx_s)

    x = jnp.where(_colmask(j, x_ref.shape, V), x_ref[...] / _TEMP, -jnp.inf)
    e = jnp.exp(x - m_ref[...])
    p = e / z_ref[...]
    kept = x > th_ref[...]
    sn = se_ref[...] / z_ref[...]
    f = jnp.where(kept, p / sn, 0.0)
    f_out[...] = f
    val = jnp.where(kept, jnp.log(f) + g_ref[...], -jnp.inf)
    lmax = jnp.max(val, axis=-1, keepdims=True)
    lidx = jnp.argmax(val, axis=-1).astype(jnp.int32)[:, None] + j * _C
    upd = lmax > best_s[...]
    bidx_s[...] = jnp.where(upd, lidx, bidx_s[...])
    best_s[...] = jnp.where(upd, lmax, best_s[...])
    id_out[...] = bidx_s[...]


@jax.jit
def kernel(logits):
    B, V = logits.shape
    nj = pl.cdiv(V, _C)
    f32 = jnp.float32

    gumbel = jax.random.gumbel(jax.random.key(1), (B, V), f32)

    x_spec = pl.BlockSpec((B, _C), lambda j: (0, j))
    col_spec = pl.BlockSpec((B, 1), lambda j: (0, 0))

    m, z = pl.pallas_call(
        functools.partial(_stats_body, V=V),
        grid=(nj,),
        in_specs=[x_spec],
        out_specs=[col_spec, col_spec],
        out_shape=[jax.ShapeDtypeStruct((B, 1), f32)] * 2,
    )(logits)

    x_spec2 = pl.BlockSpec((B, _C), lambda r, j: (0, j))
    col_spec2 = pl.BlockSpec((B, 1), lambda r, j: (0, 0))
    theta, se = pl.pallas_call(
        functools.partial(_refine_body, V=V),
        grid=(_R, nj),
        in_specs=[x_spec2, col_spec2, col_spec2],
        out_specs=[col_spec2, col_spec2],
        out_shape=[jax.ShapeDtypeStruct((B, 1), f32)] * 2,
        scratch_shapes=[
            pltpu.VMEM((B, 1), f32),
            pltpu.VMEM((B, 1), f32),
            pltpu.VMEM((B, _NT), f32),
        ],
    )(logits, m, z)

    filtered, ids = pl.pallas_call(
        functools.partial(_final_body, V=V),
        grid=(nj,),
        in_specs=[x_spec, x_spec, col_spec, col_spec, col_spec, col_spec],
        out_specs=[x_spec, col_spec],
        out_shape=[
            jax.ShapeDtypeStruct((B, V), f32),
            jax.ShapeDtypeStruct((B, 1), jnp.int32),
        ],
        scratch_shapes=[
            pltpu.VMEM((B, 1), f32),
            pltpu.VMEM((B, 1), jnp.int32),
        ],
    )(logits, gumbel, m, z, theta, se)

    return ids[:, 0], filtered


# speedup vs baseline: 44.7360x; 1.5548x over previous
import functools

import jax
import jax.numpy as jnp
from jax import lax
from jax.experimental import pallas as pl
from jax.experimental.pallas import tpu as pltpu
from jax.experimental.pallas import tpu_sc as plsc

_TEMP = 0.8
_TOPP = 0.9
_NT = 64
_R = 4
_W0 = 20.0
_C = 32768
_NB = 2048
_CK = 20000


def _colmask(j, shape, V):
    lane = jax.lax.broadcasted_iota(jnp.int32, shape, 1)
    return (j * _C + lane) < V


def _stats_body(x_ref, m_out, z_out, V):
    j = pl.program_id(0)
    x = jnp.where(_colmask(j, x_ref.shape, V), x_ref[...] / _TEMP, -jnp.inf)
    cm = jnp.max(x, axis=-1, keepdims=True)
    ce = jnp.sum(jnp.exp(x - cm), axis=-1, keepdims=True)

    @pl.when(j == 0)
    def _():
        m_out[...] = jnp.full_like(m_out, -jnp.inf)
        z_out[...] = jnp.zeros_like(z_out)

    m_old = m_out[...]
    m_new = jnp.maximum(m_old, cm)
    z_out[...] = z_out[...] * jnp.exp(m_old - m_new) + ce * jnp.exp(cm - m_new)
    m_out[...] = m_new


def _refine_body(x_ref, m_ref, z_ref, th_out, se_out, hi_s, w_s, acc_s, V):
    r = pl.program_id(0)
    j = pl.program_id(1)
    nj = pl.num_programs(1)

    @pl.when((r == 0) & (j == 0))
    def _():
        hi_s[...] = m_ref[...]
        w_s[...] = jnp.full_like(w_s, _W0 / _NT)

    @pl.when(j == 0)
    def _():
        acc_s[...] = jnp.zeros_like(acc_s)

    x = jnp.where(_colmask(j, x_ref.shape, V), x_ref[...] / _TEMP, -jnp.inf)
    e = jnp.exp(x - m_ref[...])
    hi = hi_s[...]
    w = w_s[...]
    lane = jax.lax.broadcasted_iota(jnp.int32, acc_s.shape, 1)
    inc = jnp.zeros(acc_s.shape, jnp.float32)
    for t in range(_NT):
        s_t = jnp.sum(jnp.where(x > (hi - (t + 1.0) * w), e, 0.0), axis=-1,
                      keepdims=True)
        inc = jnp.where(lane == t, s_t, inc)
    acc_s[...] += inc

    @pl.when(j == nj - 1)
    def _():
        acc = acc_s[...]
        target = _TOPP * z_ref[...]
        notc = jnp.where(acc > target, 0.0, 1.0)
        tstar_i = jnp.sum(notc, axis=-1, keepdims=True).astype(jnp.int32)
        tstar = tstar_i.astype(jnp.float32)
        th_out[...] = hi - (tstar + 1.0) * w
        lane_t = jax.lax.broadcasted_iota(jnp.int32, acc.shape, 1)
        se_out[...] = jnp.sum(
            jnp.where(lane_t == tstar_i, acc, 0.0), axis=-1, keepdims=True)
        hi_s[...] = hi - tstar * w
        w_s[...] = 2.0 * w / _NT


def _sc_refine_body(x_hbm, m_hbm, z_hbm, th_hbm, se_hbm, buf, hist, io16):
    c = lax.axis_index("c")
    s = lax.axis_index("s")
    w = s * 2 + c
    lane = lax.iota(jnp.int32, 16)
    f32 = jnp.float32

    def lanesum(v):
        for k in (1, 2, 4, 8):
            v = v + v.at[lane ^ k].get(mode="promise_in_bounds")
        return v

    pltpu.sync_copy(m_hbm.at[w], io16)
    mrow = io16[...]
    pltpu.sync_copy(z_hbm.at[w], io16)
    target_v = _TOPP * io16[...]

    hi_v = mrow
    wd_v = jnp.full((16,), _W0 / _NB, f32)
    V = x_hbm.shape[0] // m_hbm.shape[0]
    nck = V // _CK
    row0 = w * V
    theta_v = hi_v
    smin_v = jnp.zeros((16,), f32)
    zero_v = jnp.zeros((16,), f32)
    one_v = jnp.ones((16,), f32)
    inf_v = jnp.full((16,), jnp.inf, f32)

    for _ in range(2):
        def zero_body(i, _):
            hist[pl.ds(i * 16, 16)] = zero_v
            return 0
        lax.fori_loop(0, _NB, zero_body, 0)
        winv_v = one_v / wd_v
        hi_b = hi_v

        def chunk_body(i, _):
            off = pl.multiple_of(row0 + i * _CK, 8)
            pltpu.sync_copy(x_hbm.at[pl.ds(off, _CK)], buf)

            def vec_body(k, _2):
                x = buf[pl.ds(k * 16, 16)] / _TEMP
                e = jnp.exp(x - mrow)
                u = (hi_b - x) * winv_v
                b = jnp.minimum(jnp.maximum(u.astype(jnp.int32), 0), _NB - 1)
                plsc.addupdate_scatter(hist, [b * 16 + lane], e)
                return 0

            lax.fori_loop(0, _CK // 16, vec_body, 0)
            return 0

        lax.fori_loop(0, nck, chunk_body, 0)

        def scan_body(i, carry):
            run, cnt, sm = carry
            cum = run + lanesum(hist[pl.ds(i * 16, 16)])
            over = cum > target_v
            cnt = cnt + jnp.where(over, zero_v, one_v)
            sm = jnp.minimum(sm, jnp.where(over, cum, inf_v))
            return cum, cnt, sm

        _, tstar_v, smin_v = lax.fori_loop(
            0, _NB, scan_body, (zero_v, zero_v, inf_v))
        theta_v = hi_v - (tstar_v + 1.0) * wd_v
        hi_v = hi_v - tstar_v * wd_v
        wd_v = 2.0 * wd_v / _NB

    io16[...] = theta_v
    pltpu.sync_copy(io16, th_hbm.at[w])
    io16[...] = smin_v
    pltpu.sync_copy(io16, se_hbm.at[w])


def _final_body(x_ref, g_ref, m_ref, z_ref, th_ref, se_ref,
                f_out, id_out, best_s, bidx_s, V):
    j = pl.program_id(0)

    @pl.when(j == 0)
    def _():
        best_s[...] = jnp.full_like(best_s, -jnp.inf)
        bidx_s[...] = jnp.zeros_like(bidx_s)

    x = jnp.where(_colmask(j, x_ref.shape, V), x_ref[...] / _TEMP, -jnp.inf)
    e = jnp.exp(x - m_ref[...])
    p = e / z_ref[...]
    kept = x > th_ref[...]
    sn = se_ref[...] / z_ref[...]
    f = jnp.where(kept, p / sn, 0.0)
    f_out[...] = f
    val = jnp.where(kept, jnp.log(f) + g_ref[...], -jnp.inf)
    lmax = jnp.max(val, axis=-1, keepdims=True)
    lidx = jnp.argmax(val, axis=-1).astype(jnp.int32)[:, None] + j * _C
    upd = lmax > best_s[...]
    bidx_s[...] = jnp.where(upd, lidx, bidx_s[...])
    best_s[...] = jnp.where(upd, lmax, best_s[...])
    id_out[...] = bidx_s[...]


@jax.jit
def kernel(logits):
    B, V = logits.shape
    nj = pl.cdiv(V, _C)
    f32 = jnp.float32

    gumbel = jax.random.gumbel(jax.random.key(1), (B, V), f32)

    x_spec = pl.BlockSpec((B, _C), lambda j: (0, j))
    col_spec = pl.BlockSpec((B, 1), lambda j: (0, 0))

    m, z = pl.pallas_call(
        functools.partial(_stats_body, V=V),
        grid=(nj,),
        in_specs=[x_spec],
        out_specs=[col_spec, col_spec],
        out_shape=[jax.ShapeDtypeStruct((B, 1), f32)] * 2,
    )(logits)

    m16 = jnp.broadcast_to(m, (B, 16))
    z16 = jnp.broadcast_to(z, (B, 16))
    theta16, se16 = pl.kernel(
        _sc_refine_body,
        out_type=[jax.ShapeDtypeStruct((B, 16), f32)] * 2,
        mesh=plsc.VectorSubcoreMesh(core_axis_name="c", subcore_axis_name="s"),
        compiler_params=pltpu.CompilerParams(needs_layout_passes=False),
        scratch_types=[
            pltpu.VMEM((_CK,), f32),
            pltpu.VMEM((_NB * 16,), f32),
            pltpu.VMEM((16,), f32),
        ],
    )(logits.reshape(-1), m16, z16)
    theta = theta16[:, :1]
    se = se16[:, :1]

    filtered, ids = pl.pallas_call(
        functools.partial(_final_body, V=V),
        grid=(nj,),
        in_specs=[x_spec, x_spec, col_spec, col_spec, col_spec, col_spec],
        out_specs=[x_spec, col_spec],
        out_shape=[
            jax.ShapeDtypeStruct((B, V), f32),
            jax.ShapeDtypeStruct((B, 1), jnp.int32),
        ],
        scratch_shapes=[
            pltpu.VMEM((B, 1), f32),
            pltpu.VMEM((B, 1), jnp.int32),
        ],
    )(logits, gumbel, m, z, theta, se)

    return ids[:, 0], filtered


# speedup vs baseline: 48.0227x; 1.0735x over previous
import functools

import jax
import jax.numpy as jnp
from jax import lax
from jax.experimental import pallas as pl
from jax.experimental.pallas import tpu as pltpu
from jax.experimental.pallas import tpu_sc as plsc

_TEMP = 0.8
_TOPP = 0.9
_NT = 64
_R = 4
_W0 = 20.0
_C = 32768
_NB = 2048
_CK = 20000


def _colmask(j, shape, V):
    lane = jax.lax.broadcasted_iota(jnp.int32, shape, 1)
    return (j * _C + lane) < V


def _stats_body(x_ref, m_out, z_out, V):
    j = pl.program_id(0)
    x = jnp.where(_colmask(j, x_ref.shape, V), x_ref[...] / _TEMP, -jnp.inf)
    cm = jnp.max(x, axis=-1, keepdims=True)
    ce = jnp.sum(jnp.exp(x - cm), axis=-1, keepdims=True)

    @pl.when(j == 0)
    def _():
        m_out[...] = jnp.full_like(m_out, -jnp.inf)
        z_out[...] = jnp.zeros_like(z_out)

    m_old = m_out[...]
    m_new = jnp.maximum(m_old, cm)
    z_out[...] = z_out[...] * jnp.exp(m_old - m_new) + ce * jnp.exp(cm - m_new)
    m_out[...] = m_new


def _refine_body(x_ref, m_ref, z_ref, th_out, se_out, hi_s, w_s, acc_s, V):
    r = pl.program_id(0)
    j = pl.program_id(1)
    nj = pl.num_programs(1)

    @pl.when((r == 0) & (j == 0))
    def _():
        hi_s[...] = m_ref[...]
        w_s[...] = jnp.full_like(w_s, _W0 / _NT)

    @pl.when(j == 0)
    def _():
        acc_s[...] = jnp.zeros_like(acc_s)

    x = jnp.where(_colmask(j, x_ref.shape, V), x_ref[...] / _TEMP, -jnp.inf)
    e = jnp.exp(x - m_ref[...])
    hi = hi_s[...]
    w = w_s[...]
    lane = jax.lax.broadcasted_iota(jnp.int32, acc_s.shape, 1)
    inc = jnp.zeros(acc_s.shape, jnp.float32)
    for t in range(_NT):
        s_t = jnp.sum(jnp.where(x > (hi - (t + 1.0) * w), e, 0.0), axis=-1,
                      keepdims=True)
        inc = jnp.where(lane == t, s_t, inc)
    acc_s[...] += inc

    @pl.when(j == nj - 1)
    def _():
        acc = acc_s[...]
        target = _TOPP * z_ref[...]
        notc = jnp.where(acc > target, 0.0, 1.0)
        tstar_i = jnp.sum(notc, axis=-1, keepdims=True).astype(jnp.int32)
        tstar = tstar_i.astype(jnp.float32)
        th_out[...] = hi - (tstar + 1.0) * w
        lane_t = jax.lax.broadcasted_iota(jnp.int32, acc.shape, 1)
        se_out[...] = jnp.sum(
            jnp.where(lane_t == tstar_i, acc, 0.0), axis=-1, keepdims=True)
        hi_s[...] = hi - tstar * w
        w_s[...] = 2.0 * w / _NT


def _sc_refine_body(x_hbm, m_hbm, z_hbm, th_hbm, se_hbm,
                    buf0, buf1, io16, hist, sem0, sem1):
    c = lax.axis_index("c")
    s = lax.axis_index("s")
    w = s * 2 + c
    lane = lax.iota(jnp.int32, 16)
    f32 = jnp.float32

    def lanesum(v):
        for k in (1, 2, 4, 8):
            v = v + v.at[lane ^ k].get(mode="promise_in_bounds")
        return v

    pltpu.sync_copy(m_hbm.at[w], io16)
    mrow = io16[...]
    pltpu.sync_copy(z_hbm.at[w], io16)
    target_v = _TOPP * io16[...]

    hi_v = mrow
    wd_v = jnp.full((16,), _W0 / _NB, f32)
    V = x_hbm.shape[0] // m_hbm.shape[0]
    nck = V // _CK
    row0 = w * V
    theta_v = hi_v
    smin_v = jnp.zeros((16,), f32)
    zero_v = jnp.zeros((16,), f32)
    one_v = jnp.ones((16,), f32)
    inf_v = jnp.full((16,), jnp.inf, f32)
    inv_t = jnp.full((16,), 1.0 / _TEMP, f32)
    temp_v = jnp.full((16,), _TEMP, f32)
    _U = 10
    nv = _CK // (16 * _U)

    def chunk_src(i):
        off = pl.multiple_of(row0 + i * _CK, 8)
        return x_hbm.at[pl.ds(off, _CK)]

    for _ in range(2):
        def zero_body(i, _):
            hist[pl.ds(i * 16, 16)] = zero_v
            return 0
        lax.fori_loop(0, _NB, zero_body, 0)
        hi8 = hi_v * temp_v
        wi8 = one_v / wd_v * inv_t

        def process(buf):
            def vec_body(k, _2):
                for ui in range(_U):
                    lv = buf[pl.ds((k * _U + ui) * 16, 16)]
                    e = jnp.exp(lv * inv_t - mrow)
                    u = (hi8 - lv) * wi8
                    b = jnp.minimum(
                        jnp.maximum(u.astype(jnp.int32), 0), _NB - 1)
                    plsc.addupdate_scatter(
                        hist, [lax.shift_left(b, 4) + lane], e)
                return 0

            lax.fori_loop(0, nv, vec_body, 0)

        pltpu.async_copy(chunk_src(0), buf0, sem0)

        def pair_body(i2, _):
            base = i2 * 2
            pltpu.async_copy(chunk_src(base + 1), buf1, sem1)
            pltpu.make_async_copy(chunk_src(base), buf0, sem0).wait()
            process(buf0)

            @pl.when(base + 2 < nck)
            def _():
                pltpu.async_copy(chunk_src(base + 2), buf0, sem0)

            pltpu.make_async_copy(chunk_src(base + 1), buf1, sem1).wait()
            process(buf1)
            return 0

        lax.fori_loop(0, nck // 2, pair_body, 0)

        def scan_body(i, carry):
            run, cnt, sm = carry
            cum = run + lanesum(hist[pl.ds(i * 16, 16)])
            over = cum > target_v
            cnt = cnt + jnp.where(over, zero_v, one_v)
            sm = jnp.minimum(sm, jnp.where(over, cum, inf_v))
            return cum, cnt, sm

        _, tstar_v, smin_v = lax.fori_loop(
            0, _NB, scan_body, (zero_v, zero_v, inf_v))
        theta_v = hi_v - (tstar_v + 1.0) * wd_v
        hi_v = hi_v - tstar_v * wd_v
        wd_v = 2.0 * wd_v / _NB

    io16[...] = theta_v
    pltpu.sync_copy(io16, th_hbm.at[w])
    io16[...] = smin_v
    pltpu.sync_copy(io16, se_hbm.at[w])


def _final_body(x_ref, g_ref, m_ref, z_ref, th_ref, se_ref,
                f_out, id_out, best_s, bidx_s, V):
    j = pl.program_id(0)

    @pl.when(j == 0)
    def _():
        best_s[...] = jnp.full_like(best_s, -jnp.inf)
        bidx_s[...] = jnp.zeros_like(bidx_s)

    x = jnp.where(_colmask(j, x_ref.shape, V), x_ref[...] / _TEMP, -jnp.inf)
    e = jnp.exp(x - m_ref[...])
    p = e / z_ref[...]
    kept = x > th_ref[...]
    sn = se_ref[...] / z_ref[...]
    f = jnp.where(kept, p / sn, 0.0)
    f_out[...] = f
    val = jnp.where(kept, jnp.log(f) + g_ref[...], -jnp.inf)
    lmax = jnp.max(val, axis=-1, keepdims=True)
    lidx = jnp.argmax(val, axis=-1).astype(jnp.int32)[:, None] + j * _C
    upd = lmax > best_s[...]
    bidx_s[...] = jnp.where(upd, lidx, bidx_s[...])
    best_s[...] = jnp.where(upd, lmax, best_s[...])
    id_out[...] = bidx_s[...]


@jax.jit
def kernel(logits):
    B, V = logits.shape
    nj = pl.cdiv(V, _C)
    f32 = jnp.float32

    gumbel = jax.random.gumbel(jax.random.key(1), (B, V), f32)

    x_spec = pl.BlockSpec((B, _C), lambda j: (0, j))
    col_spec = pl.BlockSpec((B, 1), lambda j: (0, 0))

    m, z = pl.pallas_call(
        functools.partial(_stats_body, V=V),
        grid=(nj,),
        in_specs=[x_spec],
        out_specs=[col_spec, col_spec],
        out_shape=[jax.ShapeDtypeStruct((B, 1), f32)] * 2,
    )(logits)

    m16 = jnp.broadcast_to(m, (B, 16))
    z16 = jnp.broadcast_to(z, (B, 16))
    theta16, se16 = pl.kernel(
        _sc_refine_body,
        out_type=[jax.ShapeDtypeStruct((B, 16), f32)] * 2,
        mesh=plsc.VectorSubcoreMesh(core_axis_name="c", subcore_axis_name="s"),
        compiler_params=pltpu.CompilerParams(needs_layout_passes=False),
        scratch_types=[
            pltpu.VMEM((_CK,), f32),
            pltpu.VMEM((_CK,), f32),
            pltpu.VMEM((16,), f32),
            pltpu.VMEM((_NB * 16,), f32),
            pltpu.SemaphoreType.DMA,
            pltpu.SemaphoreType.DMA,
        ],
    )(logits.reshape(-1), m16, z16)
    theta = theta16[:, :1]
    se = se16[:, :1]

    filtered, ids = pl.pallas_call(
        functools.partial(_final_body, V=V),
        grid=(nj,),
        in_specs=[x_spec, x_spec, col_spec, col_spec, col_spec, col_spec],
        out_specs=[x_spec, col_spec],
        out_shape=[
            jax.ShapeDtypeStruct((B, V), f32),
            jax.ShapeDtypeStruct((B, 1), jnp.int32),
        ],
        scratch_shapes=[
            pltpu.VMEM((B, 1), f32),
            pltpu.VMEM((B, 1), jnp.int32),
        ],
    )(logits, gumbel, m, z, theta, se)

    return ids[:, 0], filtered


# speedup vs baseline: 48.0370x; 1.0003x over previous
import functools

import jax
import jax.numpy as jnp
from jax import lax
from jax.experimental import pallas as pl
from jax.experimental.pallas import tpu as pltpu
from jax.experimental.pallas import tpu_sc as plsc

_TEMP = 0.8
_TOPP = 0.9
_NT = 64
_R = 4
_W0 = 20.0
_C = 32768
_NB = 2048
_CK = 20000


def _colmask(j, shape, V):
    lane = jax.lax.broadcasted_iota(jnp.int32, shape, 1)
    return (j * _C + lane) < V


def _stats_body(x_ref, m_out, z_out, V):
    j = pl.program_id(0)
    x = jnp.where(_colmask(j, x_ref.shape, V), x_ref[...] / _TEMP, -jnp.inf)
    cm = jnp.max(x, axis=-1, keepdims=True)
    ce = jnp.sum(jnp.exp(x - cm), axis=-1, keepdims=True)

    @pl.when(j == 0)
    def _():
        m_out[...] = jnp.full_like(m_out, -jnp.inf)
        z_out[...] = jnp.zeros_like(z_out)

    m_old = m_out[...]
    m_new = jnp.maximum(m_old, cm)
    z_out[...] = z_out[...] * jnp.exp(m_old - m_new) + ce * jnp.exp(cm - m_new)
    m_out[...] = m_new


def _refine_body(x_ref, m_ref, z_ref, th_out, se_out, hi_s, w_s, acc_s, V):
    r = pl.program_id(0)
    j = pl.program_id(1)
    nj = pl.num_programs(1)

    @pl.when((r == 0) & (j == 0))
    def _():
        hi_s[...] = m_ref[...]
        w_s[...] = jnp.full_like(w_s, _W0 / _NT)

    @pl.when(j == 0)
    def _():
        acc_s[...] = jnp.zeros_like(acc_s)

    x = jnp.where(_colmask(j, x_ref.shape, V), x_ref[...] / _TEMP, -jnp.inf)
    e = jnp.exp(x - m_ref[...])
    hi = hi_s[...]
    w = w_s[...]
    lane = jax.lax.broadcasted_iota(jnp.int32, acc_s.shape, 1)
    inc = jnp.zeros(acc_s.shape, jnp.float32)
    for t in range(_NT):
        s_t = jnp.sum(jnp.where(x > (hi - (t + 1.0) * w), e, 0.0), axis=-1,
                      keepdims=True)
        inc = jnp.where(lane == t, s_t, inc)
    acc_s[...] += inc

    @pl.when(j == nj - 1)
    def _():
        acc = acc_s[...]
        target = _TOPP * z_ref[...]
        notc = jnp.where(acc > target, 0.0, 1.0)
        tstar_i = jnp.sum(notc, axis=-1, keepdims=True).astype(jnp.int32)
        tstar = tstar_i.astype(jnp.float32)
        th_out[...] = hi - (tstar + 1.0) * w
        lane_t = jax.lax.broadcasted_iota(jnp.int32, acc.shape, 1)
        se_out[...] = jnp.sum(
            jnp.where(lane_t == tstar_i, acc, 0.0), axis=-1, keepdims=True)
        hi_s[...] = hi - tstar * w
        w_s[...] = 2.0 * w / _NT


def _sc_refine_body(x_hbm, m_hbm, z_hbm, th_hbm, se_hbm,
                    buf0, buf1, io16, hist, sem0, sem1):
    c = lax.axis_index("c")
    s = lax.axis_index("s")
    w = s * 2 + c
    lane = lax.iota(jnp.int32, 16)
    f32 = jnp.float32

    def lanesum(v):
        for k in (1, 2, 4, 8):
            v = v + v.at[lane ^ k].get(mode="promise_in_bounds")
        return v

    pltpu.sync_copy(m_hbm.at[w], io16)
    mrow = io16[...]
    pltpu.sync_copy(z_hbm.at[w], io16)
    target_v = _TOPP * io16[...]

    hi_v = mrow
    wd_v = jnp.full((16,), _W0 / _NB, f32)
    V = x_hbm.shape[0] // m_hbm.shape[0]
    nck = V // _CK
    row0 = w * V
    theta_v = hi_v
    smin_v = jnp.zeros((16,), f32)
    zero_v = jnp.zeros((16,), f32)
    one_v = jnp.ones((16,), f32)
    inf_v = jnp.full((16,), jnp.inf, f32)
    inv_t = jnp.full((16,), 1.0 / _TEMP, f32)
    temp_v = jnp.full((16,), _TEMP, f32)
    _U = 10
    nv = _CK // (16 * _U)

    def chunk_src(i):
        off = pl.multiple_of(row0 + i * _CK, 8)
        return x_hbm.at[pl.ds(off, _CK)]

    for _ in range(2):
        def zero_body(i, _):
            hist[pl.ds(i * 16, 16)] = zero_v
            return 0
        lax.fori_loop(0, _NB, zero_body, 0)
        hi8 = hi_v * temp_v
        wi8 = one_v / wd_v * inv_t

        def process(buf):
            def vec_body(k, _2):
                for ui in range(_U):
                    lv = buf[pl.ds((k * _U + ui) * 16, 16)]
                    e = jnp.exp(lv * inv_t - mrow)
                    u = (hi8 - lv) * wi8
                    b = jnp.minimum(
                        jnp.maximum(u.astype(jnp.int32), 0), _NB - 1)
                    plsc.addupdate_scatter(
                        hist, [lax.shift_left(b, 4) + lane], e)
                return 0

            lax.fori_loop(0, nv, vec_body, 0)

        pltpu.async_copy(chunk_src(0), buf0, sem0)

        def pair_body(i2, _):
            base = i2 * 2
            pltpu.async_copy(chunk_src(base + 1), buf1, sem1)
            pltpu.make_async_copy(chunk_src(base), buf0, sem0).wait()
            process(buf0)

            @pl.when(base + 2 < nck)
            def _():
                pltpu.async_copy(chunk_src(base + 2), buf0, sem0)

            pltpu.make_async_copy(chunk_src(base + 1), buf1, sem1).wait()
            process(buf1)
            return 0

        lax.fori_loop(0, nck // 2, pair_body, 0)

        def scan_body(i, carry):
            run, cnt, sm = carry
            cum = run + lanesum(hist[pl.ds(i * 16, 16)])
            over = cum > target_v
            cnt = cnt + jnp.where(over, zero_v, one_v)
            sm = jnp.minimum(sm, jnp.where(over, cum, inf_v))
            return cum, cnt, sm

        _, tstar_v, smin_v = lax.fori_loop(
            0, _NB, scan_body, (zero_v, zero_v, inf_v))
        theta_v = hi_v - (tstar_v + 1.0) * wd_v
        hi_v = hi_v - tstar_v * wd_v
        wd_v = 2.0 * wd_v / _NB

    io16[...] = theta_v
    pltpu.sync_copy(io16, th_hbm.at[w])
    io16[...] = smin_v
    pltpu.sync_copy(io16, se_hbm.at[w])


def _final_body(x_ref, g_ref, m_ref, z_ref, th_ref, se_ref,
                f_out, id_out, best_s, bidx_s, V):
    j = pl.program_id(0)

    @pl.when(j == 0)
    def _():
        best_s[...] = jnp.full_like(best_s, -jnp.inf)
        bidx_s[...] = jnp.zeros_like(bidx_s)

    x = jnp.where(_colmask(j, x_ref.shape, V), x_ref[...] / _TEMP, -jnp.inf)
    e = jnp.exp(x - m_ref[...])
    p = e / z_ref[...]
    kept = x > th_ref[...]
    sn = se_ref[...] / z_ref[...]
    f = jnp.where(kept, p / sn, 0.0)
    f_out[...] = f
    val = jnp.where(kept, jnp.log(f) + g_ref[...], -jnp.inf)
    lmax = jnp.max(val, axis=-1, keepdims=True)
    lidx = jnp.argmax(val, axis=-1).astype(jnp.int32)[:, None] + j * _C
    upd = lmax > best_s[...]
    bidx_s[...] = jnp.where(upd, lidx, bidx_s[...])
    best_s[...] = jnp.where(upd, lmax, best_s[...])
    id_out[...] = bidx_s[...]


@jax.jit
def kernel(logits):
    B, V = logits.shape
    nj = pl.cdiv(V, _C)
    f32 = jnp.float32

    x_spec = pl.BlockSpec((B, _C), lambda j: (0, j))
    col_spec = pl.BlockSpec((B, 1), lambda j: (0, 0))

    m, z = pl.pallas_call(
        functools.partial(_stats_body, V=V),
        grid=(nj,),
        in_specs=[x_spec],
        out_specs=[col_spec, col_spec],
        out_shape=[jax.ShapeDtypeStruct((B, 1), f32)] * 2,
    )(logits)

    m16 = jnp.broadcast_to(m, (B, 16))
    z16 = jnp.broadcast_to(z, (B, 16))
    theta16, se16 = pl.kernel(
        _sc_refine_body,
        out_type=[jax.ShapeDtypeStruct((B, 16), f32)] * 2,
        mesh=plsc.VectorSubcoreMesh(core_axis_name="c", subcore_axis_name="s"),
        compiler_params=pltpu.CompilerParams(needs_layout_passes=False),
        scratch_types=[
            pltpu.VMEM((_CK,), f32),
            pltpu.VMEM((_CK,), f32),
            pltpu.VMEM((16,), f32),
            pltpu.VMEM((_NB * 16,), f32),
            pltpu.SemaphoreType.DMA,
            pltpu.SemaphoreType.DMA,
        ],
    )(logits.reshape(-1), m16, z16)
    theta = theta16[:, :1]
    se = se16[:, :1]

    gumbel = jax.random.gumbel(jax.random.key(1), (B, V), f32)

    filtered, ids = pl.pallas_call(
        functools.partial(_final_body, V=V),
        grid=(nj,),
        in_specs=[x_spec, x_spec, col_spec, col_spec, col_spec, col_spec],
        out_specs=[x_spec, col_spec],
        out_shape=[
            jax.ShapeDtypeStruct((B, V), f32),
            jax.ShapeDtypeStruct((B, 1), jnp.int32),
        ],
        scratch_shapes=[
            pltpu.VMEM((B, 1), f32),
            pltpu.VMEM((B, 1), jnp.int32),
        ],
    )(logits, gumbel, m, z, theta, se)

    return ids[:, 0], filtered
